# baseline (speedup 1.0000x reference)
import functools

import jax
import jax.numpy as jnp
from jax import lax
from jax.experimental import pallas as pl
from jax.experimental.pallas import tpu as pltpu
from jax.experimental.pallas import tpu_sc as plsc

DIM = 1024
N_EXPERTS = 64
TOP_K = 2
INTER = 256
T = 256
A = 512
SLOTS = 1024
CH = 8
EPS = 4


def _nt(a, b):
    return lax.dot_general(a, b, (((1,), (1,)), ((), ())),
                           preferred_element_type=jnp.float32)


def _route_body(x_ref, gw_ref, bias_ref, sw1_ref, sw2_ref, sw3_ref,
                xs_ref, dest_ref, wb_ref, offs_ref, pcnt_ref, sh_ref):
    xv = x_ref[...]
    scores = _nt(xv, gw_ref[...]) + bias_ref[...]
    smax = jnp.max(scores, axis=1, keepdims=True)
    ex = jnp.exp(scores - smax)
    probs = ex / jnp.sum(ex, axis=1, keepdims=True)
    idxe = lax.broadcasted_iota(jnp.int32, (T, N_EXPERTS), 1)
    big = jnp.int32(10_000)
    m1 = jnp.max(probs, axis=1, keepdims=True)
    i1 = jnp.min(jnp.where(probs >= m1, idxe, big), axis=1, keepdims=True)
    pm = jnp.where(idxe == i1, jnp.float32(-1.0), probs)
    m2 = jnp.max(pm, axis=1, keepdims=True)
    i2 = jnp.min(jnp.where(pm >= m2, idxe, big), axis=1, keepdims=True)
    wsum = m1 + m2 + jnp.float32(1e-8)
    wn1 = m1 / wsum
    wn2 = m2 / wsum

    e_col = jnp.concatenate([i1, i2], axis=0)
    w_col = jnp.concatenate([wn1, wn2], axis=0)
    iota_e = lax.broadcasted_iota(jnp.int32, (1, N_EXPERTS), 1)
    amat = (e_col == iota_e).astype(jnp.float32)
    ra = lax.broadcasted_iota(jnp.int32, (A, A), 0)
    ca = lax.broadcasted_iota(jnp.int32, (A, A), 1)
    ltri = (ca <= ra).astype(jnp.float32)
    cum = jnp.dot(ltri, amat, preferred_element_type=jnp.float32)
    rank = jnp.sum(cum * amat, axis=1, keepdims=True) - 1.0
    counts = jnp.sum(amat, axis=0, keepdims=True)
    pcnt = jnp.floor((counts + 7.0) * 0.125) * 8.0
    re = lax.broadcasted_iota(jnp.int32, (N_EXPERTS, N_EXPERTS), 0)
    ce = lax.broadcasted_iota(jnp.int32, (N_EXPERTS, N_EXPERTS), 1)
    umat = (re < ce).astype(jnp.float32)
    offs = jnp.dot(pcnt, umat, preferred_element_type=jnp.float32)
    dest = jnp.sum(amat * offs, axis=1, keepdims=True) + rank
    dest_i = dest.astype(jnp.int32)
    iota_p = lax.broadcasted_iota(jnp.int32, (1, SLOTS), 1)
    omat = (dest_i == iota_p).astype(jnp.float32)
    xx = jnp.concatenate([xv, xv], axis=0)
    xs_ref[...] = lax.dot_general(
        omat, xx, (((0,), (0,)), ((), ())),
        preferred_element_type=jnp.float32)

    dest_ref[...] = dest_i
    wb_ref[...] = jnp.broadcast_to(w_col, (A, 16))
    offs_ref[...] = offs.astype(jnp.int32)
    pcnt_ref[...] = pcnt.astype(jnp.int32)

    s1 = _nt(xv, sw1_ref[...])
    s3 = _nt(xv, sw3_ref[...])
    hs = s1 * (1.0 / (1.0 + jnp.exp(-s1))) * s3
    sh_ref[...] = _nt(hs, sw2_ref[...])


def _route(x_flat, gate_weight, bias_row, sw1, sw2, sw3):
    outs = (
        jax.ShapeDtypeStruct((SLOTS, DIM), jnp.float32),
        jax.ShapeDtypeStruct((A, 1), jnp.int32),
        jax.ShapeDtypeStruct((A, 16), jnp.float32),
        jax.ShapeDtypeStruct((1, N_EXPERTS), jnp.int32),
        jax.ShapeDtypeStruct((1, N_EXPERTS), jnp.int32),
        jax.ShapeDtypeStruct((T, DIM), jnp.float32),
    )
    return pl.pallas_call(_route_body, out_shape=outs)(
        x_flat, gate_weight, bias_row, sw1, sw2, sw3)


def _expert_body(offs_ref, pcnt_ref, x_ref, w1_ref, w3_ref, w2_ref, o_ref):
    g = pl.program_id(0)
    for i in range(EPS):
        off = offs_ref[g * EPS + i]
        cnt = pcnt_ref[g * EPS + i]
        w1b = w1_ref[i]
        w3b = w3_ref[i]
        w2b = w2_ref[i]

        def chunk(j, carry, off=off, w1b=w1b, w3b=w3b, w2b=w2b):
            base = pl.multiple_of(off + j * CH, CH)
            xs = x_ref[pl.ds(base, CH), :]
            h1 = _nt(xs, w1b)
            h3 = _nt(xs, w3b)
            h = h1 * (1.0 / (1.0 + jnp.exp(-h1))) * h3
            o_ref[pl.ds(base, CH), :] = _nt(h, w2b)
            return carry

        lax.fori_loop(0, cnt // CH, chunk, 0)


def _experts(offs, pcnt, x_sorted, w1, w3, w2):
    grid_spec = pltpu.PrefetchScalarGridSpec(
        num_scalar_prefetch=2,
        grid=(N_EXPERTS // EPS,),
        in_specs=[
            pl.BlockSpec((SLOTS, DIM), lambda g, *_: (0, 0)),
            pl.BlockSpec((EPS, INTER, DIM), lambda g, *_: (g, 0, 0)),
            pl.BlockSpec((EPS, INTER, DIM), lambda g, *_: (g, 0, 0)),
            pl.BlockSpec((EPS, DIM, INTER), lambda g, *_: (g, 0, 0)),
        ],
        out_specs=pl.BlockSpec((SLOTS, DIM), lambda g, *_: (0, 0)),
    )
    return pl.pallas_call(
        _expert_body,
        grid_spec=grid_spec,
        out_shape=jax.ShapeDtypeStruct((SLOTS, DIM), jnp.float32),
    )(offs, pcnt, x_sorted, w1, w3, w2)


def _sc_combine(out_sorted, dest, wb, shared):
    info = plsc.get_sparse_core_info()
    nw = info.num_cores * info.num_subcores
    t_per_w = T // nw
    n_idx = 2 * t_per_w
    mesh = plsc.VectorSubcoreMesh(core_axis_name="c", subcore_axis_name="s")

    @functools.partial(
        pl.kernel,
        out_type=jax.ShapeDtypeStruct((T, DIM), jnp.float32),
        mesh=mesh,
        scratch_types=[
            pltpu.VMEM((n_idx,), jnp.int32),
            pltpu.VMEM((n_idx, 16), jnp.float32),
            pltpu.VMEM((n_idx, DIM), jnp.float32),
            pltpu.VMEM((t_per_w, DIM), jnp.float32),
            pltpu.VMEM((t_per_w, DIM), jnp.float32),
            pltpu.SemaphoreType.DMA,
        ],
    )
    def k(os_hbm, dest_hbm, wb_hbm, sh_hbm, out_hbm,
          idx_v, w_v, rows_v, sh_v, acc_v, sem):
        wid = lax.axis_index("s") * info.num_cores + lax.axis_index("c")
        tb = wid * t_per_w
        pltpu.sync_copy(dest_hbm.at[pl.ds(tb, t_per_w)], idx_v.at[pl.ds(0, t_per_w)])
        pltpu.sync_copy(dest_hbm.at[pl.ds(T + tb, t_per_w)],
                        idx_v.at[pl.ds(t_per_w, t_per_w)])
        pltpu.sync_copy(wb_hbm.at[pl.ds(tb, t_per_w)], w_v.at[pl.ds(0, t_per_w)])
        pltpu.sync_copy(wb_hbm.at[pl.ds(T + tb, t_per_w)],
                        w_v.at[pl.ds(t_per_w, t_per_w)])
        pltpu.sync_copy(sh_hbm.at[pl.ds(tb, t_per_w)], sh_v)
        pltpu.async_copy(os_hbm.at[idx_v], rows_v, sem).wait()

        for t in range(t_per_w):
            w0 = w_v[t, :]
            w1l = w_v[t_per_w + t, :]

            def body(c, carry, t=t, w0=w0, w1l=w1l):
                sl = pl.ds(c * 16, 16)
                acc_v[t, sl] = (rows_v[t, sl] * w0
                                + rows_v[t_per_w + t, sl] * w1l
                                + sh_v[t, sl])
                return carry

            lax.fori_loop(0, DIM // 16, body, 0)
        pltpu.sync_copy(acc_v, out_hbm.at[pl.ds(tb, t_per_w)])

    return k(out_sorted, dest, wb, shared)


def kernel(x, gate_weight, adaptive_bias, w1, w2, w3, sw1, sw2, sw3):
    b, s, d = x.shape
    x_flat = x.reshape(-1, d)
    bias_row = adaptive_bias.reshape(1, N_EXPERTS)
    x_sorted, dest, wb, offs, pcnt, shared = _route(
        x_flat, gate_weight, bias_row, sw1, sw2, sw3)
    out_sorted = _experts(offs.reshape(N_EXPERTS), pcnt.reshape(N_EXPERTS),
                          x_sorted, w1, w3, w2)
    out = _sc_combine(out_sorted, dest.reshape(A), wb, shared)
    return out.reshape(b, s, d)


# speedup vs baseline: 1.0094x; 1.0094x over previous
import functools

import jax
import jax.numpy as jnp
from jax import lax
from jax.experimental import pallas as pl
from jax.experimental.pallas import tpu as pltpu
from jax.experimental.pallas import tpu_sc as plsc

DIM = 1024
N_EXPERTS = 64
TOP_K = 2
INTER = 256
T = 256
A = 512
SLOTS = 1024
CH = 8
EPS = 4


def _nt(a, b):
    return lax.dot_general(a, b, (((1,), (1,)), ((), ())),
                           preferred_element_type=jnp.float32)


def _route_body(x_ref, gw_ref, bias_ref, sw1_ref, sw2_ref, sw3_ref,
                xs_ref, dest_ref, wb_ref, offs_ref, pcnt_ref, sh_ref):
    xv = x_ref[...]
    scores = _nt(xv, gw_ref[...]) + bias_ref[...]
    smax = jnp.max(scores, axis=1, keepdims=True)
    ex = jnp.exp(scores - smax)
    probs = ex / jnp.sum(ex, axis=1, keepdims=True)
    idxe = lax.broadcasted_iota(jnp.int32, (T, N_EXPERTS), 1)
    big = jnp.int32(10_000)
    m1 = jnp.max(probs, axis=1, keepdims=True)
    i1 = jnp.min(jnp.where(probs >= m1, idxe, big), axis=1, keepdims=True)
    pm = jnp.where(idxe == i1, jnp.float32(-1.0), probs)
    m2 = jnp.max(pm, axis=1, keepdims=True)
    i2 = jnp.min(jnp.where(pm >= m2, idxe, big), axis=1, keepdims=True)
    wsum = m1 + m2 + jnp.float32(1e-8)
    wn1 = m1 / wsum
    wn2 = m2 / wsum

    e_col = jnp.concatenate([i1, i2], axis=0)
    w_col = jnp.concatenate([wn1, wn2], axis=0)
    iota_e = lax.broadcasted_iota(jnp.int32, (1, N_EXPERTS), 1)
    amat = (e_col == iota_e).astype(jnp.float32)
    ra = lax.broadcasted_iota(jnp.int32, (A, A), 0)
    ca = lax.broadcasted_iota(jnp.int32, (A, A), 1)
    ltri = (ca <= ra).astype(jnp.float32)
    cum = jnp.dot(ltri, amat, preferred_element_type=jnp.float32)
    rank = jnp.sum(cum * amat, axis=1, keepdims=True) - 1.0
    counts = jnp.sum(amat, axis=0, keepdims=True)
    pcnt = jnp.floor((counts + 7.0) * 0.125) * 8.0
    re = lax.broadcasted_iota(jnp.int32, (N_EXPERTS, N_EXPERTS), 0)
    ce = lax.broadcasted_iota(jnp.int32, (N_EXPERTS, N_EXPERTS), 1)
    umat = (re < ce).astype(jnp.float32)
    offs = jnp.dot(pcnt, umat, preferred_element_type=jnp.float32)
    dest = jnp.sum(amat * offs, axis=1, keepdims=True) + rank
    dest_i = dest.astype(jnp.int32)
    iota_p = lax.broadcasted_iota(jnp.int32, (1, SLOTS), 1)
    omat = (dest_i == iota_p).astype(jnp.float32)
    xx = jnp.concatenate([xv, xv], axis=0)
    xs_ref[...] = lax.dot_general(
        omat, xx, (((0,), (0,)), ((), ())),
        preferred_element_type=jnp.float32)

    dest_ref[...] = dest_i
    wb_ref[...] = jnp.broadcast_to(w_col, (A, 16))
    offs_ref[...] = offs.astype(jnp.int32)
    pcnt_ref[...] = pcnt.astype(jnp.int32)

    s1 = _nt(xv, sw1_ref[...])
    s3 = _nt(xv, sw3_ref[...])
    hs = s1 * (1.0 / (1.0 + jnp.exp(-s1))) * s3
    sh_ref[...] = _nt(hs, sw2_ref[...])


def _route(x_flat, gate_weight, bias_row, sw1, sw2, sw3):
    outs = (
        jax.ShapeDtypeStruct((SLOTS, DIM), jnp.float32),
        jax.ShapeDtypeStruct((A, 1), jnp.int32),
        jax.ShapeDtypeStruct((A, 16), jnp.float32),
        jax.ShapeDtypeStruct((1, N_EXPERTS), jnp.int32),
        jax.ShapeDtypeStruct((1, N_EXPERTS), jnp.int32),
        jax.ShapeDtypeStruct((T, DIM), jnp.float32),
    )
    return pl.pallas_call(_route_body, out_shape=outs)(
        x_flat, gate_weight, bias_row, sw1, sw2, sw3)


def _expert_body(offs_ref, pcnt_ref, x_ref, w1_ref, w3_ref, w2_ref, o_ref):
    g = pl.program_id(0)
    for i in range(EPS):
        off = offs_ref[g * EPS + i]
        cnt = pcnt_ref[g * EPS + i]
        w1b = w1_ref[i]
        w3b = w3_ref[i]
        w2b = w2_ref[i]

        def chunk(j, carry, off=off, w1b=w1b, w3b=w3b, w2b=w2b):
            base = pl.multiple_of(off + j * CH, CH)
            xs = x_ref[pl.ds(base, CH), :]
            h1 = _nt(xs, w1b)
            h3 = _nt(xs, w3b)
            h = h1 * (1.0 / (1.0 + jnp.exp(-h1))) * h3
            o_ref[pl.ds(base, CH), :] = _nt(h, w2b)
            return carry

        lax.fori_loop(0, cnt // CH, chunk, 0)


def _experts(offs, pcnt, x_sorted, w1, w3, w2):
    grid_spec = pltpu.PrefetchScalarGridSpec(
        num_scalar_prefetch=2,
        grid=(N_EXPERTS // EPS,),
        in_specs=[
            pl.BlockSpec((SLOTS, DIM), lambda g, *_: (0, 0)),
            pl.BlockSpec((EPS, INTER, DIM), lambda g, *_: (g, 0, 0)),
            pl.BlockSpec((EPS, INTER, DIM), lambda g, *_: (g, 0, 0)),
            pl.BlockSpec((EPS, DIM, INTER), lambda g, *_: (g, 0, 0)),
        ],
        out_specs=pl.BlockSpec((SLOTS, DIM), lambda g, *_: (0, 0)),
    )
    return pl.pallas_call(
        _expert_body,
        grid_spec=grid_spec,
        out_shape=jax.ShapeDtypeStruct((SLOTS, DIM), jnp.float32),
    )(offs, pcnt, x_sorted, w1, w3, w2)


def _sc_combine(out_sorted, dest, wb, shared):
    info = plsc.get_sparse_core_info()
    nw = info.num_cores * info.num_subcores
    t_per_w = T // nw
    n_idx = 2 * t_per_w
    mesh = plsc.VectorSubcoreMesh(core_axis_name="c", subcore_axis_name="s")

    @functools.partial(
        pl.kernel,
        out_type=jax.ShapeDtypeStruct((T, DIM), jnp.float32),
        mesh=mesh,
        scratch_types=[
            pltpu.VMEM((n_idx,), jnp.int32),
            pltpu.VMEM((n_idx, 16), jnp.float32),
            pltpu.VMEM((n_idx, DIM), jnp.float32),
            pltpu.VMEM((t_per_w, DIM), jnp.float32),
            pltpu.VMEM((t_per_w, DIM), jnp.float32),
            pltpu.SemaphoreType.DMA,
        ],
    )
    def k(os_hbm, dest_hbm, wb_hbm, sh_hbm, out_hbm,
          idx_v, w_v, rows_v, sh_v, acc_v, sem):
        wid = lax.axis_index("s") * info.num_cores + lax.axis_index("c")
        tb = wid * t_per_w
        pltpu.sync_copy(dest_hbm.at[pl.ds(tb, t_per_w)], idx_v.at[pl.ds(0, t_per_w)])
        pltpu.sync_copy(dest_hbm.at[pl.ds(T + tb, t_per_w)],
                        idx_v.at[pl.ds(t_per_w, t_per_w)])
        gather = pltpu.async_copy(os_hbm.at[idx_v], rows_v, sem)
        pltpu.sync_copy(wb_hbm.at[pl.ds(tb, t_per_w)], w_v.at[pl.ds(0, t_per_w)])
        pltpu.sync_copy(wb_hbm.at[pl.ds(T + tb, t_per_w)],
                        w_v.at[pl.ds(t_per_w, t_per_w)])
        pltpu.sync_copy(sh_hbm.at[pl.ds(tb, t_per_w)], sh_v)
        gather.wait()

        for t in range(t_per_w):
            w0 = w_v[t, :]
            w1l = w_v[t_per_w + t, :]

            def body(c, carry, t=t, w0=w0, w1l=w1l):
                sl = pl.ds(c * 16, 16)
                acc_v[t, sl] = (rows_v[t, sl] * w0
                                + rows_v[t_per_w + t, sl] * w1l
                                + sh_v[t, sl])
                return carry

            lax.fori_loop(0, DIM // 16, body, 0)
        pltpu.sync_copy(acc_v, out_hbm.at[pl.ds(tb, t_per_w)])

    return k(out_sorted, dest, wb, shared)


def kernel(x, gate_weight, adaptive_bias, w1, w2, w3, sw1, sw2, sw3):
    b, s, d = x.shape
    x_flat = x.reshape(-1, d)
    bias_row = adaptive_bias.reshape(1, N_EXPERTS)
    x_sorted, dest, wb, offs, pcnt, shared = _route(
        x_flat, gate_weight, bias_row, sw1, sw2, sw3)
    out_sorted = _experts(offs.reshape(N_EXPERTS), pcnt.reshape(N_EXPERTS),
                          x_sorted, w1, w3, w2)
    out = _sc_combine(out_sorted, dest.reshape(A), wb, shared)
    return out.reshape(b, s, d)


# speedup vs baseline: 1.0139x; 1.0045x over previous
import functools

import jax
import jax.numpy as jnp
from jax import lax
from jax.experimental import pallas as pl
from jax.experimental.pallas import tpu as pltpu
from jax.experimental.pallas import tpu_sc as plsc

DIM = 1024
N_EXPERTS = 64
TOP_K = 2
INTER = 256
T = 256
A = 512
SLOTS = 1024
CH = 8
EPS = 4


def _nt(a, b):
    return lax.dot_general(a, b, (((1,), (1,)), ((), ())),
                           preferred_element_type=jnp.float32)


def _route_body(x_ref, gw_ref, bias_ref, sw1_ref, sw2_ref, sw3_ref,
                om_ref, dest_ref, wb_ref, offs_ref, pcnt_ref, sh_ref):
    xv = x_ref[...]
    scores = _nt(xv, gw_ref[...]) + bias_ref[...]
    smax = jnp.max(scores, axis=1, keepdims=True)
    ex = jnp.exp(scores - smax)
    probs = ex / jnp.sum(ex, axis=1, keepdims=True)
    idxe = lax.broadcasted_iota(jnp.int32, (T, N_EXPERTS), 1)
    big = jnp.int32(10_000)
    m1 = jnp.max(probs, axis=1, keepdims=True)
    i1 = jnp.min(jnp.where(probs >= m1, idxe, big), axis=1, keepdims=True)
    pm = jnp.where(idxe == i1, jnp.float32(-1.0), probs)
    m2 = jnp.max(pm, axis=1, keepdims=True)
    i2 = jnp.min(jnp.where(pm >= m2, idxe, big), axis=1, keepdims=True)
    wsum = m1 + m2 + jnp.float32(1e-8)
    wn1 = m1 / wsum
    wn2 = m2 / wsum

    e_col = jnp.concatenate([i1, i2], axis=0)
    w_col = jnp.concatenate([wn1, wn2], axis=0)
    iota_e = lax.broadcasted_iota(jnp.int32, (1, N_EXPERTS), 1)
    amat = (e_col == iota_e).astype(jnp.float32)
    ra = lax.broadcasted_iota(jnp.int32, (A, A), 0)
    ca = lax.broadcasted_iota(jnp.int32, (A, A), 1)
    ltri = (ca <= ra).astype(jnp.float32)
    cum = jnp.dot(ltri, amat, preferred_element_type=jnp.float32)
    rank = jnp.sum(cum * amat, axis=1, keepdims=True) - 1.0
    counts = jnp.sum(amat, axis=0, keepdims=True)
    pcnt = jnp.floor((counts + 7.0) * 0.125) * 8.0
    re = lax.broadcasted_iota(jnp.int32, (N_EXPERTS, N_EXPERTS), 0)
    ce = lax.broadcasted_iota(jnp.int32, (N_EXPERTS, N_EXPERTS), 1)
    umat = (re < ce).astype(jnp.float32)
    offs = jnp.dot(pcnt, umat, preferred_element_type=jnp.float32)
    dest = jnp.sum(amat * offs, axis=1, keepdims=True) + rank
    dest_i = dest.astype(jnp.int32)
    iota_p = lax.broadcasted_iota(jnp.int32, (1, SLOTS), 1)
    omat = (dest_i == iota_p).astype(jnp.float32)
    om_ref[...] = omat

    dest_ref[...] = dest_i
    wb_ref[...] = jnp.broadcast_to(w_col, (A, 16))
    offs_ref[...] = offs.astype(jnp.int32)
    pcnt_ref[...] = pcnt.astype(jnp.int32)

    s1 = _nt(xv, sw1_ref[...])
    s3 = _nt(xv, sw3_ref[...])
    hs = s1 * (1.0 / (1.0 + jnp.exp(-s1))) * s3
    sh_ref[...] = _nt(hs, sw2_ref[...])


def _route(x_flat, gate_weight, bias_row, sw1, sw2, sw3):
    outs = (
        jax.ShapeDtypeStruct((A, SLOTS), jnp.float32),
        jax.ShapeDtypeStruct((A, 1), jnp.int32),
        jax.ShapeDtypeStruct((A, 16), jnp.float32),
        jax.ShapeDtypeStruct((1, N_EXPERTS), jnp.int32),
        jax.ShapeDtypeStruct((1, N_EXPERTS), jnp.int32),
        jax.ShapeDtypeStruct((T, DIM), jnp.float32),
    )
    return pl.pallas_call(_route_body, out_shape=outs)(
        x_flat, gate_weight, bias_row, sw1, sw2, sw3)


def _expert_body(offs_ref, pcnt_ref, om_ref, x_ref, w1_ref, w3_ref, w2_ref,
                 o_ref, xs_scr):
    g = pl.program_id(0)

    @pl.when(g == 0)
    def _dispatch():
        xv = x_ref[...]
        xx = jnp.concatenate([xv, xv], axis=0)
        xs_scr[...] = lax.dot_general(
            om_ref[...], xx, (((0,), (0,)), ((), ())),
            preferred_element_type=jnp.float32)

    for i in range(EPS):
        off = offs_ref[g * EPS + i]
        cnt = pcnt_ref[g * EPS + i]
        w1b = w1_ref[i]
        w3b = w3_ref[i]
        w2b = w2_ref[i]

        def chunk(j, carry, off=off, w1b=w1b, w3b=w3b, w2b=w2b):
            base = pl.multiple_of(off + j * CH, CH)
            xs = xs_scr[pl.ds(base, CH), :]
            h1 = _nt(xs, w1b)
            h3 = _nt(xs, w3b)
            h = h1 * (1.0 / (1.0 + jnp.exp(-h1))) * h3
            o_ref[pl.ds(base, CH), :] = _nt(h, w2b)
            return carry

        lax.fori_loop(0, cnt // CH, chunk, 0)


def _experts(offs, pcnt, omat, x_flat, w1, w3, w2):
    grid_spec = pltpu.PrefetchScalarGridSpec(
        num_scalar_prefetch=2,
        grid=(N_EXPERTS // EPS,),
        in_specs=[
            pl.BlockSpec((A, SLOTS), lambda g, *_: (0, 0)),
            pl.BlockSpec((T, DIM), lambda g, *_: (0, 0)),
            pl.BlockSpec((EPS, INTER, DIM), lambda g, *_: (g, 0, 0)),
            pl.BlockSpec((EPS, INTER, DIM), lambda g, *_: (g, 0, 0)),
            pl.BlockSpec((EPS, DIM, INTER), lambda g, *_: (g, 0, 0)),
        ],
        out_specs=pl.BlockSpec((SLOTS, DIM), lambda g, *_: (0, 0)),
        scratch_shapes=[pltpu.VMEM((SLOTS, DIM), jnp.float32)],
    )
    return pl.pallas_call(
        _expert_body,
        grid_spec=grid_spec,
        out_shape=jax.ShapeDtypeStruct((SLOTS, DIM), jnp.float32),
    )(offs, pcnt, omat, x_flat, w1, w3, w2)


def _sc_combine(out_sorted, dest, wb, shared):
    info = plsc.get_sparse_core_info()
    nw = info.num_cores * info.num_subcores
    t_per_w = T // nw
    n_idx = 2 * t_per_w
    mesh = plsc.VectorSubcoreMesh(core_axis_name="c", subcore_axis_name="s")

    @functools.partial(
        pl.kernel,
        out_type=jax.ShapeDtypeStruct((T, DIM), jnp.float32),
        mesh=mesh,
        scratch_types=[
            pltpu.VMEM((n_idx,), jnp.int32),
            pltpu.VMEM((n_idx, 16), jnp.float32),
            pltpu.VMEM((n_idx, DIM), jnp.float32),
            pltpu.VMEM((t_per_w, DIM), jnp.float32),
            pltpu.VMEM((t_per_w, DIM), jnp.float32),
            pltpu.SemaphoreType.DMA,
        ],
    )
    def k(os_hbm, dest_hbm, wb_hbm, sh_hbm, out_hbm,
          idx_v, w_v, rows_v, sh_v, acc_v, sem):
        wid = lax.axis_index("s") * info.num_cores + lax.axis_index("c")
        tb = wid * t_per_w
        pltpu.sync_copy(dest_hbm.at[pl.ds(tb, t_per_w)], idx_v.at[pl.ds(0, t_per_w)])
        pltpu.sync_copy(dest_hbm.at[pl.ds(T + tb, t_per_w)],
                        idx_v.at[pl.ds(t_per_w, t_per_w)])
        gather = pltpu.async_copy(os_hbm.at[idx_v], rows_v, sem)
        pltpu.sync_copy(wb_hbm.at[pl.ds(tb, t_per_w)], w_v.at[pl.ds(0, t_per_w)])
        pltpu.sync_copy(wb_hbm.at[pl.ds(T + tb, t_per_w)],
                        w_v.at[pl.ds(t_per_w, t_per_w)])
        pltpu.sync_copy(sh_hbm.at[pl.ds(tb, t_per_w)], sh_v)
        gather.wait()

        for t in range(t_per_w):
            w0 = w_v[t, :]
            w1l = w_v[t_per_w + t, :]

            def body(c, carry, t=t, w0=w0, w1l=w1l):
                sl = pl.ds(c * 16, 16)
                acc_v[t, sl] = (rows_v[t, sl] * w0
                                + rows_v[t_per_w + t, sl] * w1l
                                + sh_v[t, sl])
                return carry

            lax.fori_loop(0, DIM // 16, body, 0)
        pltpu.sync_copy(acc_v, out_hbm.at[pl.ds(tb, t_per_w)])

    return k(out_sorted, dest, wb, shared)


def kernel(x, gate_weight, adaptive_bias, w1, w2, w3, sw1, sw2, sw3):
    b, s, d = x.shape
    x_flat = x.reshape(-1, d)
    bias_row = adaptive_bias.reshape(1, N_EXPERTS)
    omat, dest, wb, offs, pcnt, shared = _route(
        x_flat, gate_weight, bias_row, sw1, sw2, sw3)
    out_sorted = _experts(offs.reshape(N_EXPERTS), pcnt.reshape(N_EXPERTS),
                          omat, x_flat, w1, w3, w2)
    out = _sc_combine(out_sorted, dest.reshape(A), wb, shared)
    return out.reshape(b, s, d)
